# baseline (speedup 1.0000x reference)
import functools

import jax
import jax.numpy as jnp
from jax import lax
from jax.experimental import pallas as pl
from jax.experimental.pallas import tpu as pltpu
from jax.experimental.pallas import tpu_sc as plsc

B = 16384
F = 26
V = 1000000
D = 13

NC = 2
NS = 16
NW = NC * NS
EPW = B // NW
LANE = 16
IPW = EPW * F
DPW = EPW * D

_mesh = plsc.VectorSubcoreMesh(
    core_axis_name="c", subcore_axis_name="s", num_cores=NC, num_subcores=NS
)


@functools.partial(
    pl.kernel,
    out_type=jax.ShapeDtypeStruct((B,), jnp.float32),
    mesh=_mesh,
    scratch_types=[
        pltpu.VMEM((IPW,), jnp.int32),
        pltpu.VMEM((IPW,), jnp.int32),
        pltpu.VMEM((IPW,), jnp.int32),
        pltpu.VMEM((IPW,), jnp.float32),
        pltpu.VMEM((DPW,), jnp.int32),
        pltpu.VMEM((DPW,), jnp.float32),
        pltpu.VMEM((EPW,), jnp.float32),
        pltpu.VMEM((D, LANE), jnp.float32),
        pltpu.VMEM((LANE,), jnp.float32),
        pltpu.SemaphoreType.DMA,
        pltpu.SemaphoreType.DMA,
    ],
)
def _sc_call(ids_hbm, dense_hbm, flat_hbm, pc_hbm, oc_hbm, pd_hbm, wd_hbm,
             bias_hbm, out_hbm,
             pc_v, oc_v, idx_v, vals_v, pd_v, dv_v, out_v, wd_v, bias_v,
             sem, semd):
    wid = lax.axis_index("s") * NC + lax.axis_index("c")

    pltpu.sync_copy(pc_hbm, pc_v)
    pltpu.sync_copy(pd_hbm, pd_v)
    pltpu.sync_copy(oc_hbm, oc_v)
    pltpu.sync_copy(wd_hbm, wd_v)
    pltpu.sync_copy(bias_hbm, bias_v)

    def _abs_pc(i, _):
        sl = pl.ds(i * LANE, LANE)
        pc_v[sl] = pc_v[sl] + wid * IPW
        return 0

    lax.fori_loop(0, IPW // LANE, _abs_pc, 0)

    def _abs_pd(i, _):
        sl = pl.ds(i * LANE, LANE)
        pd_v[sl] = pd_v[sl] + wid * DPW
        return 0

    lax.fori_loop(0, DPW // LANE, _abs_pd, 0)

    cp_ids = pltpu.async_copy(ids_hbm.at[pc_v], idx_v, sem)
    cp_den = pltpu.async_copy(dense_hbm.at[pd_v], dv_v, semd)
    cp_ids.wait()

    def _off_chunk(i, _):
        sl = pl.ds(i * LANE, LANE)
        idx_v[sl] = idx_v[sl] + oc_v[sl]
        return 0

    lax.fori_loop(0, IPW // LANE, _off_chunk, 0)

    pltpu.async_copy(flat_hbm.at[idx_v], vals_v, sem).wait()
    cp_den.wait()

    wsplat = [wd_v[d, :] for d in range(D)]
    bias_splat = bias_v[...]

    def _chunk(t, _):
        def _fsum(f, acc):
            return acc + vals_v[pl.ds(f * EPW + t * LANE, LANE)]

        acc = lax.fori_loop(0, F, _fsum, jnp.zeros((LANE,), jnp.float32))
        for d in range(D):
            acc = acc + dv_v[pl.ds(d * EPW + t * LANE, LANE)] * wsplat[d]
        acc = acc + bias_splat
        out_v[pl.ds(t * LANE, LANE)] = 1.0 / (1.0 + jnp.exp(-acc))
        return 0

    lax.fori_loop(0, EPW // LANE, _chunk, 0)

    pltpu.sync_copy(out_v, out_hbm.at[pl.ds(wid * EPW, EPW)])


def kernel(sparse_ids, dense_features, W_cat, W_dense, bias):
    ids = sparse_ids.astype(jnp.int32).reshape(B * F)
    dense_w = dense_features.reshape(B * D)
    flat_w = W_cat.reshape(-1)
    q = jnp.arange(IPW, dtype=jnp.int32)
    pc = (q % EPW) * F + q // EPW
    oc = (q // EPW) * V
    qd = jnp.arange(DPW, dtype=jnp.int32)
    pd = (qd % EPW) * D + qd // EPW
    wd_bc = jnp.broadcast_to(W_dense.reshape(D, 1), (D, LANE))
    bias_bc = jnp.broadcast_to(bias.reshape(1), (LANE,))
    out = _sc_call(ids, dense_w, flat_w, pc, oc, pd, wd_bc, bias_bc)
    return out.reshape(B, 1)


# speedup vs baseline: 12.0990x; 12.0990x over previous
import functools

import jax
import jax.numpy as jnp
from jax import lax
from jax.experimental import pallas as pl
from jax.experimental.pallas import tpu as pltpu
from jax.experimental.pallas import tpu_sc as plsc

B = 16384
F = 26
V = 1000000
D = 13
VP = 1 << 20
VA = 999936
VT = V - VA
TAIL = F * VP

NC = 2
NS = 16
NW = NC * NS
EPW = B // NW
LANE = 16
IPW = EPW * F
DPW = EPW * D

NBUF = 4


def _flatten_body(w_ref, tail_ref, out_ref, buf, sem_in, sem_out, sem_t):
    pltpu.make_async_copy(tail_ref, out_ref.at[pl.ds(TAIL, F * VT)],
                          sem_t).start()

    def _in_copy(f):
        return pltpu.make_async_copy(w_ref.at[f], buf.at[f % NBUF],
                                     sem_in.at[f % NBUF])

    def _out_copy(f):
        return pltpu.make_async_copy(buf.at[f % NBUF].at[pl.ds(0, VA)],
                                     out_ref.at[pl.ds(f * VP, VA)],
                                     sem_out.at[f % NBUF])

    for f in range(NBUF):
        _in_copy(f).start()
    for f in range(F):
        _in_copy(f).wait()
        _out_copy(f).start()
        if f + NBUF < F:
            _out_copy(f).wait()
            _in_copy(f + NBUF).start()
    for f in range(F - NBUF, F):
        _out_copy(f).wait()
    pltpu.make_async_copy(tail_ref, out_ref.at[pl.ds(TAIL, F * VT)],
                          sem_t).wait()


def _flatten_table(w_cat, tail_flat):
    return pl.pallas_call(
        _flatten_body,
        in_specs=[pl.BlockSpec(memory_space=pltpu.MemorySpace.HBM),
                  pl.BlockSpec(memory_space=pltpu.MemorySpace.HBM)],
        out_specs=pl.BlockSpec(memory_space=pltpu.MemorySpace.HBM),
        out_shape=jax.ShapeDtypeStruct((F * VP + F * VT,), jnp.float32),
        scratch_shapes=[
            pltpu.VMEM((NBUF, V), jnp.float32),
            pltpu.SemaphoreType.DMA((NBUF,)),
            pltpu.SemaphoreType.DMA((NBUF,)),
            pltpu.SemaphoreType.DMA,
        ],
    )(w_cat, tail_flat)


_mesh = plsc.VectorSubcoreMesh(
    core_axis_name="c", subcore_axis_name="s", num_cores=NC, num_subcores=NS
)


@functools.partial(
    pl.kernel,
    out_type=jax.ShapeDtypeStruct((B,), jnp.float32),
    mesh=_mesh,
    scratch_types=[
        pltpu.VMEM((IPW,), jnp.int32),
        pltpu.VMEM((IPW,), jnp.int32),
        pltpu.VMEM((IPW,), jnp.int32),
        pltpu.VMEM((IPW,), jnp.int32),
        pltpu.VMEM((IPW,), jnp.float32),
        pltpu.VMEM((DPW,), jnp.int32),
        pltpu.VMEM((DPW,), jnp.float32),
        pltpu.VMEM((EPW,), jnp.float32),
        pltpu.VMEM((D, LANE), jnp.float32),
        pltpu.VMEM((LANE,), jnp.float32),
        pltpu.SemaphoreType.DMA,
        pltpu.SemaphoreType.DMA,
    ],
)
def _sc_call(ids_hbm, dense_hbm, flat_hbm, pc_hbm, oc_hbm, ot_hbm, pd_hbm,
             wd_hbm, bias_hbm, out_hbm,
             pc_v, oc_v, ot_v, idx_v, vals_v, pd_v, dv_v, out_v, wd_v, bias_v,
             sem, semd):
    wid = lax.axis_index("s") * NC + lax.axis_index("c")

    pltpu.sync_copy(pc_hbm, pc_v)
    pltpu.sync_copy(pd_hbm, pd_v)
    pltpu.sync_copy(oc_hbm, oc_v)
    pltpu.sync_copy(ot_hbm, ot_v)
    pltpu.sync_copy(wd_hbm, wd_v)
    pltpu.sync_copy(bias_hbm, bias_v)

    def _abs_pc(i, _):
        sl = pl.ds(i * LANE, LANE)
        pc_v[sl] = pc_v[sl] + wid * IPW
        return 0

    lax.fori_loop(0, IPW // LANE, _abs_pc, 0)

    def _abs_pd(i, _):
        sl = pl.ds(i * LANE, LANE)
        pd_v[sl] = pd_v[sl] + wid * DPW
        return 0

    lax.fori_loop(0, DPW // LANE, _abs_pd, 0)

    cp_ids = pltpu.async_copy(ids_hbm.at[pc_v], idx_v, sem)
    cp_den = pltpu.async_copy(dense_hbm.at[pd_v], dv_v, semd)
    cp_ids.wait()

    def _off_chunk(i, _):
        sl = pl.ds(i * LANE, LANE)
        raw = idx_v[sl]
        idx_v[sl] = jnp.where(raw < VA, raw + oc_v[sl], raw + ot_v[sl])
        return 0

    lax.fori_loop(0, IPW // LANE, _off_chunk, 0)

    pltpu.async_copy(flat_hbm.at[idx_v], vals_v, sem).wait()
    cp_den.wait()

    wsplat = [wd_v[d, :] for d in range(D)]
    bias_splat = bias_v[...]

    def _chunk(t, _):
        def _fsum(f, acc):
            return acc + vals_v[pl.ds(f * EPW + t * LANE, LANE)]

        acc = lax.fori_loop(0, F, _fsum, jnp.zeros((LANE,), jnp.float32))
        for d in range(D):
            acc = acc + dv_v[pl.ds(d * EPW + t * LANE, LANE)] * wsplat[d]
        acc = acc + bias_splat
        out_v[pl.ds(t * LANE, LANE)] = 1.0 / (1.0 + jnp.exp(-acc))
        return 0

    lax.fori_loop(0, EPW // LANE, _chunk, 0)

    pltpu.sync_copy(out_v, out_hbm.at[pl.ds(wid * EPW, EPW)])


def kernel(sparse_ids, dense_features, W_cat, W_dense, bias):
    ids = sparse_ids.astype(jnp.int32).reshape(B * F)
    dense_w = dense_features.reshape(B * D)
    tail_flat = W_cat[:, VA:].reshape(F * VT)
    flat_w = _flatten_table(W_cat, tail_flat)
    q = jnp.arange(IPW, dtype=jnp.int32)
    pc = (q % EPW) * F + q // EPW
    oc = (q // EPW) * VP
    ot = TAIL - VA + (q // EPW) * VT
    qd = jnp.arange(DPW, dtype=jnp.int32)
    pd = (qd % EPW) * D + qd // EPW
    wd_bc = jnp.broadcast_to(W_dense.reshape(D, 1), (D, LANE))
    bias_bc = jnp.broadcast_to(bias.reshape(1), (LANE,))
    out = _sc_call(ids, dense_w, flat_w, pc, oc, ot, pd, wd_bc, bias_bc)
    return out.reshape(B, 1)


# speedup vs baseline: 13.2728x; 1.0970x over previous
import functools

import jax
import jax.numpy as jnp
from jax import lax
from jax.experimental import pallas as pl
from jax.experimental.pallas import tpu as pltpu
from jax.experimental.pallas import tpu_sc as plsc

B = 16384
F = 26
V = 1000000
D = 13
VP = 1 << 20
VA = 999936
VT = V - VA
TAIL = F * VP

NC = 2
NS = 16
NW = NC * NS
EPW = B // NW
LANE = 16
IPW = EPW * F
DPW = EPW * D

NBUF = 4


def _flatten_body(w_ref, tail_ref, out_ref, buf, sem_in, sem_out, sem_t):
    pltpu.make_async_copy(tail_ref, out_ref.at[pl.ds(TAIL, F * VT)],
                          sem_t).start()

    def _in_copy(f):
        return pltpu.make_async_copy(w_ref.at[f], buf.at[f % NBUF],
                                     sem_in.at[f % NBUF])

    def _out_copy(f):
        return pltpu.make_async_copy(buf.at[f % NBUF].at[pl.ds(0, VA)],
                                     out_ref.at[pl.ds(f * VP, VA)],
                                     sem_out.at[f % NBUF])

    for f in range(NBUF):
        _in_copy(f).start()
    for f in range(F):
        _in_copy(f).wait()
        _out_copy(f).start()
        if f + NBUF < F:
            _out_copy(f).wait()
            _in_copy(f + NBUF).start()
    for f in range(F - NBUF, F):
        _out_copy(f).wait()
    pltpu.make_async_copy(tail_ref, out_ref.at[pl.ds(TAIL, F * VT)],
                          sem_t).wait()


def _flatten_table(w_cat, tail_flat):
    return pl.pallas_call(
        _flatten_body,
        in_specs=[pl.BlockSpec(memory_space=pltpu.MemorySpace.HBM),
                  pl.BlockSpec(memory_space=pltpu.MemorySpace.HBM)],
        out_specs=pl.BlockSpec(memory_space=pltpu.MemorySpace.HBM),
        out_shape=jax.ShapeDtypeStruct((F * VP + F * VT,), jnp.float32),
        scratch_shapes=[
            pltpu.VMEM((NBUF, V), jnp.float32),
            pltpu.SemaphoreType.DMA((NBUF,)),
            pltpu.SemaphoreType.DMA((NBUF,)),
            pltpu.SemaphoreType.DMA,
        ],
    )(w_cat, tail_flat)


_mesh = plsc.VectorSubcoreMesh(
    core_axis_name="c", subcore_axis_name="s", num_cores=NC, num_subcores=NS
)


@functools.partial(
    pl.kernel,
    out_type=(jax.ShapeDtypeStruct((B * F,), jnp.int32),
              jax.ShapeDtypeStruct((B * D,), jnp.float32)),
    mesh=_mesh,
    scratch_types=[
        pltpu.VMEM((IPW,), jnp.int32),
        pltpu.VMEM((IPW,), jnp.int32),
        pltpu.VMEM((IPW,), jnp.int32),
        pltpu.VMEM((IPW,), jnp.int32),
        pltpu.VMEM((DPW,), jnp.int32),
        pltpu.VMEM((DPW,), jnp.float32),
        pltpu.SemaphoreType.DMA,
        pltpu.SemaphoreType.DMA,
    ],
)
def _sc_prep(ids_hbm, dense_hbm, pc_hbm, oc_hbm, ot_hbm, pd_hbm,
             idx_out, dv_out,
             pc_v, oc_v, ot_v, idx_v, pd_v, dv_v, sem, semd):
    wid = lax.axis_index("s") * NC + lax.axis_index("c")

    pltpu.sync_copy(pc_hbm, pc_v)
    pltpu.sync_copy(pd_hbm, pd_v)
    pltpu.sync_copy(oc_hbm, oc_v)
    pltpu.sync_copy(ot_hbm, ot_v)

    def _abs_pc(i, _):
        sl = pl.ds(i * LANE, LANE)
        pc_v[sl] = pc_v[sl] + wid * IPW
        return 0

    lax.fori_loop(0, IPW // LANE, _abs_pc, 0)

    def _abs_pd(i, _):
        sl = pl.ds(i * LANE, LANE)
        pd_v[sl] = pd_v[sl] + wid * DPW
        return 0

    lax.fori_loop(0, DPW // LANE, _abs_pd, 0)

    cp_ids = pltpu.async_copy(ids_hbm.at[pc_v], idx_v, sem)
    cp_den = pltpu.async_copy(dense_hbm.at[pd_v], dv_v, semd)
    cp_ids.wait()

    def _off_chunk(i, _):
        sl = pl.ds(i * LANE, LANE)
        raw = idx_v[sl]
        idx_v[sl] = jnp.where(raw < VA, raw + oc_v[sl], raw + ot_v[sl])
        return 0

    lax.fori_loop(0, IPW // LANE, _off_chunk, 0)

    pltpu.sync_copy(idx_v, idx_out.at[pl.ds(wid * IPW, IPW)])
    cp_den.wait()
    pltpu.sync_copy(dv_v, dv_out.at[pl.ds(wid * DPW, DPW)])


@functools.partial(
    pl.kernel,
    out_type=jax.ShapeDtypeStruct((B,), jnp.float32),
    mesh=_mesh,
    scratch_types=[
        pltpu.VMEM((IPW,), jnp.int32),
        pltpu.VMEM((IPW,), jnp.float32),
        pltpu.VMEM((DPW,), jnp.float32),
        pltpu.VMEM((EPW,), jnp.float32),
        pltpu.VMEM((D, LANE), jnp.float32),
        pltpu.VMEM((LANE,), jnp.float32),
        pltpu.SemaphoreType.DMA,
        pltpu.SemaphoreType.DMA,
    ],
)
def _sc_main(flat_hbm, idx_hbm, dv_hbm, wd_hbm, bias_hbm, out_hbm,
             idx_v, vals_v, dv_v, out_v, wd_v, bias_v, sem, semd):
    wid = lax.axis_index("s") * NC + lax.axis_index("c")

    pltpu.sync_copy(idx_hbm.at[pl.ds(wid * IPW, IPW)], idx_v)
    cp_den = pltpu.async_copy(dv_hbm.at[pl.ds(wid * DPW, DPW)], dv_v, semd)
    pltpu.sync_copy(wd_hbm, wd_v)
    pltpu.sync_copy(bias_hbm, bias_v)

    pltpu.async_copy(flat_hbm.at[idx_v], vals_v, sem).wait()
    cp_den.wait()

    wsplat = [wd_v[d, :] for d in range(D)]
    bias_splat = bias_v[...]

    def _chunk(t, _):
        def _fsum(f, acc):
            return acc + vals_v[pl.ds(f * EPW + t * LANE, LANE)]

        acc = lax.fori_loop(0, F, _fsum, jnp.zeros((LANE,), jnp.float32))
        for d in range(D):
            acc = acc + dv_v[pl.ds(d * EPW + t * LANE, LANE)] * wsplat[d]
        acc = acc + bias_splat
        out_v[pl.ds(t * LANE, LANE)] = 1.0 / (1.0 + jnp.exp(-acc))
        return 0

    lax.fori_loop(0, EPW // LANE, _chunk, 0)

    pltpu.sync_copy(out_v, out_hbm.at[pl.ds(wid * EPW, EPW)])


def kernel(sparse_ids, dense_features, W_cat, W_dense, bias):
    ids = sparse_ids.astype(jnp.int32).reshape(B * F)
    dense_w = dense_features.reshape(B * D)
    tail_flat = W_cat[:, VA:].reshape(F * VT)
    flat_w = _flatten_table(W_cat, tail_flat)
    q = jnp.arange(IPW, dtype=jnp.int32)
    pc = (q % EPW) * F + q // EPW
    oc = (q // EPW) * VP
    ot = TAIL - VA + (q // EPW) * VT
    qd = jnp.arange(DPW, dtype=jnp.int32)
    pd = (qd % EPW) * D + qd // EPW
    wd_bc = jnp.broadcast_to(W_dense.reshape(D, 1), (D, LANE))
    bias_bc = jnp.broadcast_to(bias.reshape(1), (LANE,))
    idx_all, dv_all = _sc_prep(ids, dense_w, pc, oc, ot, pd)
    out = _sc_main(flat_w, idx_all, dv_all, wd_bc, bias_bc)
    return out.reshape(B, 1)


# speedup vs baseline: 13.3774x; 1.0079x over previous
import functools

import jax
import jax.numpy as jnp
from jax import lax
from jax.experimental import pallas as pl
from jax.experimental.pallas import tpu as pltpu
from jax.experimental.pallas import tpu_sc as plsc

B = 16384
F = 26
V = 1000000
D = 13
VP = 1 << 20
VA = 999936
VT = V - VA
TAIL = F * VP

NC = 2
NS = 16
NW = NC * NS
EPW = B // NW
LANE = 16
IPW = EPW * F
DPW = EPW * D

NBUF = 8
PRE = 4


def _flatten_body(w_ref, tail_ref, out_ref, buf, sem_in, sem_out, sem_t):
    pltpu.make_async_copy(tail_ref, out_ref.at[pl.ds(TAIL, F * VT)],
                          sem_t).start()

    def _in_copy(f):
        return pltpu.make_async_copy(w_ref.at[f], buf.at[f % NBUF],
                                     sem_in.at[f % NBUF])

    def _out_copy(f):
        return pltpu.make_async_copy(buf.at[f % NBUF].at[pl.ds(0, VA)],
                                     out_ref.at[pl.ds(f * VP, VA)],
                                     sem_out.at[f % NBUF])

    for f in range(PRE):
        _in_copy(f).start()
    for f in range(F):
        _in_copy(f).wait()
        _out_copy(f).start()
        g = f + PRE
        if g < F:
            if g >= NBUF:
                _out_copy(g - NBUF).wait()
            _in_copy(g).start()
    for f in range(F - NBUF, F):
        _out_copy(f).wait()
    pltpu.make_async_copy(tail_ref, out_ref.at[pl.ds(TAIL, F * VT)],
                          sem_t).wait()


def _flatten_table(w_cat, tail_flat):
    return pl.pallas_call(
        _flatten_body,
        in_specs=[pl.BlockSpec(memory_space=pltpu.MemorySpace.HBM),
                  pl.BlockSpec(memory_space=pltpu.MemorySpace.HBM)],
        out_specs=pl.BlockSpec(memory_space=pltpu.MemorySpace.HBM),
        out_shape=jax.ShapeDtypeStruct((F * VP + F * VT,), jnp.float32),
        scratch_shapes=[
            pltpu.VMEM((NBUF, V), jnp.float32),
            pltpu.SemaphoreType.DMA((NBUF,)),
            pltpu.SemaphoreType.DMA((NBUF,)),
            pltpu.SemaphoreType.DMA,
        ],
    )(w_cat, tail_flat)


_mesh = plsc.VectorSubcoreMesh(
    core_axis_name="c", subcore_axis_name="s", num_cores=NC, num_subcores=NS
)


@functools.partial(
    pl.kernel,
    out_type=(jax.ShapeDtypeStruct((B * F,), jnp.int32),
              jax.ShapeDtypeStruct((B * D,), jnp.float32)),
    mesh=_mesh,
    scratch_types=[
        pltpu.VMEM((IPW,), jnp.int32),
        pltpu.VMEM((IPW,), jnp.int32),
        pltpu.VMEM((IPW,), jnp.int32),
        pltpu.VMEM((IPW,), jnp.int32),
        pltpu.VMEM((DPW,), jnp.int32),
        pltpu.VMEM((DPW,), jnp.float32),
        pltpu.SemaphoreType.DMA,
        pltpu.SemaphoreType.DMA,
    ],
)
def _sc_prep(ids_hbm, dense_hbm, pc_hbm, oc_hbm, ot_hbm, pd_hbm,
             idx_out, dv_out,
             pc_v, oc_v, ot_v, idx_v, pd_v, dv_v, sem, semd):
    wid = lax.axis_index("s") * NC + lax.axis_index("c")

    pltpu.sync_copy(pc_hbm.at[wid], pc_v)
    pltpu.sync_copy(pd_hbm.at[wid], pd_v)
    pltpu.sync_copy(oc_hbm, oc_v)
    pltpu.sync_copy(ot_hbm, ot_v)

    cp_ids = pltpu.async_copy(ids_hbm.at[pc_v], idx_v, sem)
    cp_den = pltpu.async_copy(dense_hbm.at[pd_v], dv_v, semd)
    cp_ids.wait()

    def _off_chunk(i, _):
        sl = pl.ds(i * LANE, LANE)
        raw = idx_v[sl]
        idx_v[sl] = jnp.where(raw < VA, raw + oc_v[sl], raw + ot_v[sl])
        return 0

    lax.fori_loop(0, IPW // LANE, _off_chunk, 0)

    pltpu.sync_copy(idx_v, idx_out.at[pl.ds(wid * IPW, IPW)])
    cp_den.wait()
    pltpu.sync_copy(dv_v, dv_out.at[pl.ds(wid * DPW, DPW)])


@functools.partial(
    pl.kernel,
    out_type=jax.ShapeDtypeStruct((B,), jnp.float32),
    mesh=_mesh,
    scratch_types=[
        pltpu.VMEM((IPW,), jnp.int32),
        pltpu.VMEM((IPW,), jnp.float32),
        pltpu.VMEM((DPW,), jnp.float32),
        pltpu.VMEM((EPW,), jnp.float32),
        pltpu.VMEM((D, LANE), jnp.float32),
        pltpu.VMEM((LANE,), jnp.float32),
        pltpu.SemaphoreType.DMA,
        pltpu.SemaphoreType.DMA,
    ],
)
def _sc_main(flat_hbm, idx_hbm, dv_hbm, wd_hbm, bias_hbm, out_hbm,
             idx_v, vals_v, dv_v, out_v, wd_v, bias_v, sem, semd):
    wid = lax.axis_index("s") * NC + lax.axis_index("c")

    pltpu.sync_copy(idx_hbm.at[pl.ds(wid * IPW, IPW)], idx_v)
    cp_den = pltpu.async_copy(dv_hbm.at[pl.ds(wid * DPW, DPW)], dv_v, semd)
    pltpu.sync_copy(wd_hbm, wd_v)
    pltpu.sync_copy(bias_hbm, bias_v)

    pltpu.async_copy(flat_hbm.at[idx_v], vals_v, sem).wait()
    cp_den.wait()

    wsplat = [wd_v[d, :] for d in range(D)]
    bias_splat = bias_v[...]

    def _chunk(t, _):
        def _fsum(f, acc):
            return acc + vals_v[pl.ds(f * EPW + t * LANE, LANE)]

        acc = lax.fori_loop(0, F, _fsum, jnp.zeros((LANE,), jnp.float32))
        for d in range(D):
            acc = acc + dv_v[pl.ds(d * EPW + t * LANE, LANE)] * wsplat[d]
        acc = acc + bias_splat
        out_v[pl.ds(t * LANE, LANE)] = 1.0 / (1.0 + jnp.exp(-acc))
        return 0

    lax.fori_loop(0, EPW // LANE, _chunk, 0)

    pltpu.sync_copy(out_v, out_hbm.at[pl.ds(wid * EPW, EPW)])


def kernel(sparse_ids, dense_features, W_cat, W_dense, bias):
    ids = sparse_ids.astype(jnp.int32).reshape(B * F)
    dense_w = dense_features.reshape(B * D)
    tail_flat = W_cat[:, VA:].reshape(F * VT)
    flat_w = _flatten_table(W_cat, tail_flat)
    q = jnp.arange(IPW, dtype=jnp.int32)
    w = jnp.arange(NW, dtype=jnp.int32)
    pc = (w * IPW)[:, None] + ((q % EPW) * F + q // EPW)[None, :]
    oc = (q // EPW) * VP
    ot = TAIL - VA + (q // EPW) * VT
    qd = jnp.arange(DPW, dtype=jnp.int32)
    pd = (w * DPW)[:, None] + ((qd % EPW) * D + qd // EPW)[None, :]
    wd_bc = jnp.broadcast_to(W_dense.reshape(D, 1), (D, LANE))
    bias_bc = jnp.broadcast_to(bias.reshape(1), (LANE,))
    idx_all, dv_all = _sc_prep(ids, dense_w, pc, oc, ot, pd)
    out = _sc_main(flat_w, idx_all, dv_all, wd_bc, bias_bc)
    return out.reshape(B, 1)


# speedup vs baseline: 14.8418x; 1.1095x over previous
import functools

import jax
import jax.numpy as jnp
from jax import lax
from jax.experimental import pallas as pl
from jax.experimental.pallas import tpu as pltpu
from jax.experimental.pallas import tpu_sc as plsc

B = 16384
F = 26
V = 1000000
D = 13
VP = 1 << 20
VA = 999936
VT = V - VA
TAIL = F * VP

NC = 2
NS = 16
NW = NC * NS
EPW = B // NW
LANE = 16
IPW = EPW * F
DPW = EPW * D

NBUF = 8
PRE = 4
SBUF = 4


def _flatten_body(w_ref, tail_ref, ids_ref, den_ref,
                  out_ref, idsf_ref, denf_ref,
                  buf, sbuf_i, sbuf_d, sem_in, sem_out, sem_t,
                  sem_si, sem_so, sem_di, sem_do):
    def _row_ring(src, dst, sbuf, sem_i, sem_o, n_rows, width):
        def _in(r):
            return pltpu.make_async_copy(
                src.at[r].at[0], sbuf.at[pl.ds((r % SBUF) * width, width)],
                sem_i.at[r % SBUF])

        def _out(r):
            return pltpu.make_async_copy(
                sbuf.at[pl.ds((r % SBUF) * width, width)],
                dst.at[pl.ds(r * width, width)],
                sem_o.at[r % SBUF])

        for r in range(min(SBUF, n_rows)):
            _in(r).start()
        for r in range(n_rows):
            _in(r).wait()
            _out(r).start()
            if r + SBUF < n_rows:
                _out(r).wait()
                _in(r + SBUF).start()
        for r in range(max(0, n_rows - SBUF), n_rows):
            _out(r).wait()

    _row_ring(ids_ref, idsf_ref, sbuf_i, sem_si, sem_so, F, B)
    _row_ring(den_ref, denf_ref, sbuf_d, sem_di, sem_do, D, B)

    pltpu.make_async_copy(tail_ref, out_ref.at[pl.ds(TAIL, F * VT)],
                          sem_t).start()

    def _in_copy(f):
        return pltpu.make_async_copy(w_ref.at[f], buf.at[f % NBUF],
                                     sem_in.at[f % NBUF])

    def _out_copy(f):
        return pltpu.make_async_copy(buf.at[f % NBUF].at[pl.ds(0, VA)],
                                     out_ref.at[pl.ds(f * VP, VA)],
                                     sem_out.at[f % NBUF])

    for f in range(PRE):
        _in_copy(f).start()
    for f in range(F):
        _in_copy(f).wait()
        _out_copy(f).start()
        g = f + PRE
        if g < F:
            if g >= NBUF:
                _out_copy(g - NBUF).wait()
            _in_copy(g).start()
    for f in range(F - NBUF, F):
        _out_copy(f).wait()
    pltpu.make_async_copy(tail_ref, out_ref.at[pl.ds(TAIL, F * VT)],
                          sem_t).wait()


def _flatten_all(w_cat, tail_flat, ids_t, dense_t):
    return pl.pallas_call(
        _flatten_body,
        in_specs=[pl.BlockSpec(memory_space=pltpu.MemorySpace.HBM)] * 4,
        out_specs=[pl.BlockSpec(memory_space=pltpu.MemorySpace.HBM)] * 3,
        out_shape=(jax.ShapeDtypeStruct((F * VP + F * VT,), jnp.float32),
                   jax.ShapeDtypeStruct((F * B,), jnp.int32),
                   jax.ShapeDtypeStruct((D * B,), jnp.float32)),
        scratch_shapes=[
            pltpu.VMEM((NBUF, V), jnp.float32),
            pltpu.VMEM((SBUF * B,), jnp.int32),
            pltpu.VMEM((SBUF * B,), jnp.float32),
            pltpu.SemaphoreType.DMA((NBUF,)),
            pltpu.SemaphoreType.DMA((NBUF,)),
            pltpu.SemaphoreType.DMA,
            pltpu.SemaphoreType.DMA((SBUF,)),
            pltpu.SemaphoreType.DMA((SBUF,)),
            pltpu.SemaphoreType.DMA((SBUF,)),
            pltpu.SemaphoreType.DMA((SBUF,)),
        ],
    )(w_cat, tail_flat, ids_t, dense_t)


_mesh = plsc.VectorSubcoreMesh(
    core_axis_name="c", subcore_axis_name="s", num_cores=NC, num_subcores=NS
)


@functools.partial(
    pl.kernel,
    out_type=jax.ShapeDtypeStruct((B,), jnp.float32),
    mesh=_mesh,
    scratch_types=[
        pltpu.VMEM((IPW,), jnp.int32),
        pltpu.VMEM((IPW,), jnp.int32),
        pltpu.VMEM((IPW,), jnp.int32),
        pltpu.VMEM((IPW,), jnp.float32),
        pltpu.VMEM((DPW,), jnp.float32),
        pltpu.VMEM((EPW,), jnp.float32),
        pltpu.VMEM((D, LANE), jnp.float32),
        pltpu.VMEM((LANE,), jnp.float32),
        pltpu.SemaphoreType.DMA,
        pltpu.SemaphoreType.DMA,
    ],
)
def _sc_main(idsf_hbm, denf_hbm, flat_hbm, oc_hbm, ot_hbm, wd_hbm, bias_hbm,
             out_hbm, idx_v, oc_v, ot_v, vals_v, dv_v, out_v, wd_v, bias_v,
             sem, semd):
    wid = lax.axis_index("s") * NC + lax.axis_index("c")

    id_cps = [
        pltpu.async_copy(idsf_hbm.at[pl.ds(f * B + wid * EPW, EPW)],
                         idx_v.at[pl.ds(f * EPW, EPW)], sem)
        for f in range(F)
    ]
    dv_cps = [
        pltpu.async_copy(denf_hbm.at[pl.ds(d * B + wid * EPW, EPW)],
                         dv_v.at[pl.ds(d * EPW, EPW)], semd)
        for d in range(D)
    ]
    pltpu.sync_copy(oc_hbm, oc_v)
    pltpu.sync_copy(ot_hbm, ot_v)
    pltpu.sync_copy(wd_hbm, wd_v)
    pltpu.sync_copy(bias_hbm, bias_v)
    for cp in id_cps:
        cp.wait()

    def _off_chunk(i, _):
        sl = pl.ds(i * LANE, LANE)
        raw = idx_v[sl]
        idx_v[sl] = jnp.where(raw < VA, raw + oc_v[sl], raw + ot_v[sl])
        return 0

    lax.fori_loop(0, IPW // LANE, _off_chunk, 0)

    pltpu.async_copy(flat_hbm.at[idx_v], vals_v, sem).wait()
    for cp in dv_cps:
        cp.wait()

    wsplat = [wd_v[d, :] for d in range(D)]
    bias_splat = bias_v[...]

    def _chunk(t, _):
        def _fsum(f, acc):
            return acc + vals_v[pl.ds(f * EPW + t * LANE, LANE)]

        acc = lax.fori_loop(0, F, _fsum, jnp.zeros((LANE,), jnp.float32))
        for d in range(D):
            acc = acc + dv_v[pl.ds(d * EPW + t * LANE, LANE)] * wsplat[d]
        acc = acc + bias_splat
        out_v[pl.ds(t * LANE, LANE)] = 1.0 / (1.0 + jnp.exp(-acc))
        return 0

    lax.fori_loop(0, EPW // LANE, _chunk, 0)

    pltpu.sync_copy(out_v, out_hbm.at[pl.ds(wid * EPW, EPW)])


def kernel(sparse_ids, dense_features, W_cat, W_dense, bias):
    ids_t = sparse_ids.astype(jnp.int32).T.reshape(F, 1, B)
    dense_t = dense_features.T.reshape(D, 1, B)
    tail_flat = W_cat[:, VA:].reshape(F * VT)
    flat_w, ids_fm, dense_fm = _flatten_all(W_cat, tail_flat, ids_t, dense_t)
    q = jnp.arange(IPW, dtype=jnp.int32)
    oc = (q // EPW) * VP
    ot = TAIL - VA + (q // EPW) * VT
    wd_bc = jnp.broadcast_to(W_dense.reshape(D, 1), (D, LANE))
    bias_bc = jnp.broadcast_to(bias.reshape(1), (LANE,))
    out = _sc_main(ids_fm, dense_fm, flat_w, oc, ot, wd_bc, bias_bc)
    return out.reshape(B, 1)


# speedup vs baseline: 17.6208x; 1.1872x over previous
import functools

import jax
import jax.numpy as jnp
from jax import lax
from jax.experimental import pallas as pl
from jax.experimental.pallas import tpu as pltpu
from jax.experimental.pallas import tpu_sc as plsc

B = 16384
F = 26
V = 1000000
D = 13
VP = 1 << 20
VA = 999936
VT = V - VA
TAIL = F * VP

NC = 2
NS = 16
NW = NC * NS
EPW = B // NW
LANE = 16
IPW = EPW * F
DPW = EPW * D

NBUF = 8
PRE = 4
SBUF = 4


def _flatten_body(w_ref, tail_ref, ids_ref, den_ref,
                  out_ref, idsf_ref, denf_ref,
                  buf, sbuf_i, sbuf_d, sem_in, sem_out, sem_t,
                  sem_si, sem_so, sem_di, sem_do):
    def _row_in(src, sbuf, sem_i, r, width):
        return pltpu.make_async_copy(
            src.at[r].at[0], sbuf.at[pl.ds(r * width, width)], sem_i)

    def _row_out(sbuf, dst, sem_o, r, width):
        return pltpu.make_async_copy(
            sbuf.at[pl.ds(r * width, width)],
            dst.at[pl.ds(r * width, width)], sem_o)

    for r in range(F):
        _row_in(ids_ref, sbuf_i, sem_si, r, B).start()
    for r in range(D):
        _row_in(den_ref, sbuf_d, sem_di, r, B).start()

    pltpu.make_async_copy(tail_ref, out_ref.at[pl.ds(TAIL, F * VT)],
                          sem_t).start()

    def _in_copy(f):
        return pltpu.make_async_copy(w_ref.at[f], buf.at[f % NBUF],
                                     sem_in.at[f % NBUF])

    def _out_copy(f):
        return pltpu.make_async_copy(buf.at[f % NBUF].at[pl.ds(0, VA)],
                                     out_ref.at[pl.ds(f * VP, VA)],
                                     sem_out.at[f % NBUF])

    for f in range(PRE):
        _in_copy(f).start()
    for f in range(F):
        _in_copy(f).wait()
        _out_copy(f).start()
        if f < F:
            _row_in(ids_ref, sbuf_i, sem_si, f, B).wait()
            _row_out(sbuf_i, idsf_ref, sem_so, f, B).start()
        if f < D:
            _row_in(den_ref, sbuf_d, sem_di, f, B).wait()
            _row_out(sbuf_d, denf_ref, sem_do, f, B).start()
        g = f + PRE
        if g < F:
            if g >= NBUF:
                _out_copy(g - NBUF).wait()
            _in_copy(g).start()
    for f in range(F - NBUF, F):
        _out_copy(f).wait()
    for f in range(F):
        _row_out(sbuf_i, idsf_ref, sem_so, f, B).wait()
    for f in range(D):
        _row_out(sbuf_d, denf_ref, sem_do, f, B).wait()
    pltpu.make_async_copy(tail_ref, out_ref.at[pl.ds(TAIL, F * VT)],
                          sem_t).wait()


def _flatten_all(w_cat, tail_flat, ids_t, dense_t):
    return pl.pallas_call(
        _flatten_body,
        in_specs=[pl.BlockSpec(memory_space=pltpu.MemorySpace.HBM)] * 4,
        out_specs=[pl.BlockSpec(memory_space=pltpu.MemorySpace.HBM)] * 3,
        out_shape=(jax.ShapeDtypeStruct((F * VP + F * VT,), jnp.float32),
                   jax.ShapeDtypeStruct((F * B,), jnp.int32),
                   jax.ShapeDtypeStruct((D * B,), jnp.float32)),
        scratch_shapes=[
            pltpu.VMEM((NBUF, V), jnp.float32),
            pltpu.VMEM((F * B,), jnp.int32),
            pltpu.VMEM((D * B,), jnp.float32),
            pltpu.SemaphoreType.DMA((NBUF,)),
            pltpu.SemaphoreType.DMA((NBUF,)),
            pltpu.SemaphoreType.DMA,
            pltpu.SemaphoreType.DMA,
            pltpu.SemaphoreType.DMA,
            pltpu.SemaphoreType.DMA,
            pltpu.SemaphoreType.DMA,
        ],
    )(w_cat, tail_flat, ids_t, dense_t)


_mesh = plsc.VectorSubcoreMesh(
    core_axis_name="c", subcore_axis_name="s", num_cores=NC, num_subcores=NS
)


@functools.partial(
    pl.kernel,
    out_type=jax.ShapeDtypeStruct((B,), jnp.float32),
    mesh=_mesh,
    scratch_types=[
        pltpu.VMEM((IPW,), jnp.int32),
        pltpu.VMEM((IPW,), jnp.int32),
        pltpu.VMEM((IPW,), jnp.int32),
        pltpu.VMEM((IPW,), jnp.float32),
        pltpu.VMEM((DPW,), jnp.float32),
        pltpu.VMEM((EPW,), jnp.float32),
        pltpu.VMEM((D, LANE), jnp.float32),
        pltpu.VMEM((LANE,), jnp.float32),
        pltpu.SemaphoreType.DMA,
        pltpu.SemaphoreType.DMA,
    ],
)
def _sc_main(idsf_hbm, denf_hbm, flat_hbm, oc_hbm, ot_hbm, wd_hbm, bias_hbm,
             out_hbm, idx_v, oc_v, ot_v, vals_v, dv_v, out_v, wd_v, bias_v,
             sem, semd):
    wid = lax.axis_index("s") * NC + lax.axis_index("c")

    id_cps = [
        pltpu.async_copy(idsf_hbm.at[pl.ds(f * B + wid * EPW, EPW)],
                         idx_v.at[pl.ds(f * EPW, EPW)], sem)
        for f in range(F)
    ]
    dv_cps = [
        pltpu.async_copy(denf_hbm.at[pl.ds(d * B + wid * EPW, EPW)],
                         dv_v.at[pl.ds(d * EPW, EPW)], semd)
        for d in range(D)
    ]
    pltpu.sync_copy(oc_hbm, oc_v)
    pltpu.sync_copy(ot_hbm, ot_v)
    pltpu.sync_copy(wd_hbm, wd_v)
    pltpu.sync_copy(bias_hbm, bias_v)
    for cp in id_cps:
        cp.wait()

    def _off_chunk(i, _):
        sl = pl.ds(i * LANE, LANE)
        raw = idx_v[sl]
        idx_v[sl] = jnp.where(raw < VA, raw + oc_v[sl], raw + ot_v[sl])
        return 0

    lax.fori_loop(0, IPW // LANE, _off_chunk, 0)

    pltpu.async_copy(flat_hbm.at[idx_v], vals_v, sem).wait()
    for cp in dv_cps:
        cp.wait()

    wsplat = [wd_v[d, :] for d in range(D)]
    bias_splat = bias_v[...]

    def _chunk(t, _):
        def _fsum(f, acc):
            return acc + vals_v[pl.ds(f * EPW + t * LANE, LANE)]

        acc = lax.fori_loop(0, F, _fsum, jnp.zeros((LANE,), jnp.float32))
        for d in range(D):
            acc = acc + dv_v[pl.ds(d * EPW + t * LANE, LANE)] * wsplat[d]
        acc = acc + bias_splat
        out_v[pl.ds(t * LANE, LANE)] = 1.0 / (1.0 + jnp.exp(-acc))
        return 0

    lax.fori_loop(0, EPW // LANE, _chunk, 0)

    pltpu.sync_copy(out_v, out_hbm.at[pl.ds(wid * EPW, EPW)])


def kernel(sparse_ids, dense_features, W_cat, W_dense, bias):
    ids_t = sparse_ids.astype(jnp.int32).T.reshape(F, 1, B)
    dense_t = dense_features.T.reshape(D, 1, B)
    tail_flat = W_cat[:, VA:].reshape(F * VT)
    flat_w, ids_fm, dense_fm = _flatten_all(W_cat, tail_flat, ids_t, dense_t)
    q = jnp.arange(IPW, dtype=jnp.int32)
    oc = (q // EPW) * VP
    ot = TAIL - VA + (q // EPW) * VT
    wd_bc = jnp.broadcast_to(W_dense.reshape(D, 1), (D, LANE))
    bias_bc = jnp.broadcast_to(bias.reshape(1), (LANE,))
    out = _sc_main(ids_fm, dense_fm, flat_w, oc, ot, wd_bc, bias_bc)
    return out.reshape(B, 1)


# speedup vs baseline: 17.6353x; 1.0008x over previous
import functools

import jax
import jax.numpy as jnp
from jax import lax
from jax.experimental import pallas as pl
from jax.experimental.pallas import tpu as pltpu
from jax.experimental.pallas import tpu_sc as plsc

B = 16384
F = 26
V = 1000000
D = 13
VP = 1 << 20
VA = 999936
VT = V - VA
TAIL = F * VP

NC = 2
NS = 16
NW = NC * NS
EPW = B // NW
LANE = 16
IPW = EPW * F
DPW = EPW * D

NBUF = 8
PRE = 6
SBUF = 4


def _flatten_body(w_ref, tail_ref, ids_ref, den_ref,
                  out_ref, idsf_ref, denf_ref,
                  buf, sbuf_i, sbuf_d, sem_in, sem_out, sem_t,
                  sem_si, sem_so, sem_di, sem_do):
    def _row_in(src, sbuf, sem_i, r, width):
        return pltpu.make_async_copy(
            src.at[r].at[0], sbuf.at[pl.ds(r * width, width)], sem_i)

    def _row_out(sbuf, dst, sem_o, r, width):
        return pltpu.make_async_copy(
            sbuf.at[pl.ds(r * width, width)],
            dst.at[pl.ds(r * width, width)], sem_o)

    for r in range(F):
        _row_in(ids_ref, sbuf_i, sem_si, r, B).start()
    for r in range(D):
        _row_in(den_ref, sbuf_d, sem_di, r, B).start()

    pltpu.make_async_copy(tail_ref, out_ref.at[pl.ds(TAIL, F * VT)],
                          sem_t).start()

    def _in_copy(f):
        return pltpu.make_async_copy(w_ref.at[f], buf.at[f % NBUF],
                                     sem_in.at[f % NBUF])

    def _out_copy(f):
        return pltpu.make_async_copy(buf.at[f % NBUF].at[pl.ds(0, VA)],
                                     out_ref.at[pl.ds(f * VP, VA)],
                                     sem_out.at[f % NBUF])

    for f in range(PRE):
        _in_copy(f).start()
    for f in range(F):
        _in_copy(f).wait()
        _out_copy(f).start()
        if f < F:
            _row_in(ids_ref, sbuf_i, sem_si, f, B).wait()
            _row_out(sbuf_i, idsf_ref, sem_so, f, B).start()
        if f < D:
            _row_in(den_ref, sbuf_d, sem_di, f, B).wait()
            _row_out(sbuf_d, denf_ref, sem_do, f, B).start()
        g = f + PRE
        if g < F:
            if g >= NBUF:
                _out_copy(g - NBUF).wait()
            _in_copy(g).start()
    for f in range(F - NBUF, F):
        _out_copy(f).wait()
    for f in range(F):
        _row_out(sbuf_i, idsf_ref, sem_so, f, B).wait()
    for f in range(D):
        _row_out(sbuf_d, denf_ref, sem_do, f, B).wait()
    pltpu.make_async_copy(tail_ref, out_ref.at[pl.ds(TAIL, F * VT)],
                          sem_t).wait()


def _flatten_all(w_cat, tail_flat, ids_t, dense_t):
    return pl.pallas_call(
        _flatten_body,
        in_specs=[pl.BlockSpec(memory_space=pltpu.MemorySpace.HBM)] * 4,
        out_specs=[pl.BlockSpec(memory_space=pltpu.MemorySpace.HBM)] * 3,
        out_shape=(jax.ShapeDtypeStruct((F * VP + F * VT,), jnp.float32),
                   jax.ShapeDtypeStruct((F * B,), jnp.int32),
                   jax.ShapeDtypeStruct((D * B,), jnp.float32)),
        scratch_shapes=[
            pltpu.VMEM((NBUF, V), jnp.float32),
            pltpu.VMEM((F * B,), jnp.int32),
            pltpu.VMEM((D * B,), jnp.float32),
            pltpu.SemaphoreType.DMA((NBUF,)),
            pltpu.SemaphoreType.DMA((NBUF,)),
            pltpu.SemaphoreType.DMA,
            pltpu.SemaphoreType.DMA,
            pltpu.SemaphoreType.DMA,
            pltpu.SemaphoreType.DMA,
            pltpu.SemaphoreType.DMA,
        ],
    )(w_cat, tail_flat, ids_t, dense_t)


_mesh = plsc.VectorSubcoreMesh(
    core_axis_name="c", subcore_axis_name="s", num_cores=NC, num_subcores=NS
)


@functools.partial(
    pl.kernel,
    out_type=jax.ShapeDtypeStruct((B,), jnp.float32),
    mesh=_mesh,
    scratch_types=[
        pltpu.VMEM((IPW,), jnp.int32),
        pltpu.VMEM((IPW,), jnp.int32),
        pltpu.VMEM((IPW,), jnp.int32),
        pltpu.VMEM((IPW,), jnp.float32),
        pltpu.VMEM((DPW,), jnp.float32),
        pltpu.VMEM((EPW,), jnp.float32),
        pltpu.VMEM((D, LANE), jnp.float32),
        pltpu.VMEM((LANE,), jnp.float32),
        pltpu.SemaphoreType.DMA,
        pltpu.SemaphoreType.DMA,
    ],
)
def _sc_main(idsf_hbm, denf_hbm, flat_hbm, oc_hbm, ot_hbm, wd_hbm, bias_hbm,
             out_hbm, idx_v, oc_v, ot_v, vals_v, dv_v, out_v, wd_v, bias_v,
             sem, semd):
    wid = lax.axis_index("s") * NC + lax.axis_index("c")

    id_cps = [
        pltpu.async_copy(idsf_hbm.at[pl.ds(f * B + wid * EPW, EPW)],
                         idx_v.at[pl.ds(f * EPW, EPW)], sem)
        for f in range(F)
    ]
    dv_cps = [
        pltpu.async_copy(denf_hbm.at[pl.ds(d * B + wid * EPW, EPW)],
                         dv_v.at[pl.ds(d * EPW, EPW)], semd)
        for d in range(D)
    ]
    pltpu.sync_copy(oc_hbm, oc_v)
    pltpu.sync_copy(ot_hbm, ot_v)
    pltpu.sync_copy(wd_hbm, wd_v)
    pltpu.sync_copy(bias_hbm, bias_v)
    for cp in id_cps:
        cp.wait()

    def _off_chunk(i, _):
        sl = pl.ds(i * LANE, LANE)
        raw = idx_v[sl]
        idx_v[sl] = jnp.where(raw < VA, raw + oc_v[sl], raw + ot_v[sl])
        return 0

    lax.fori_loop(0, IPW // LANE, _off_chunk, 0)

    pltpu.async_copy(flat_hbm.at[idx_v], vals_v, sem).wait()
    for cp in dv_cps:
        cp.wait()

    wsplat = [wd_v[d, :] for d in range(D)]
    bias_splat = bias_v[...]

    def _chunk(t, _):
        def _fsum(f, acc):
            return acc + vals_v[pl.ds(f * EPW + t * LANE, LANE)]

        acc = lax.fori_loop(0, F, _fsum, jnp.zeros((LANE,), jnp.float32))
        for d in range(D):
            acc = acc + dv_v[pl.ds(d * EPW + t * LANE, LANE)] * wsplat[d]
        acc = acc + bias_splat
        out_v[pl.ds(t * LANE, LANE)] = 1.0 / (1.0 + jnp.exp(-acc))
        return 0

    lax.fori_loop(0, EPW // LANE, _chunk, 0)

    pltpu.sync_copy(out_v, out_hbm.at[pl.ds(wid * EPW, EPW)])


def kernel(sparse_ids, dense_features, W_cat, W_dense, bias):
    ids_t = sparse_ids.astype(jnp.int32).T.reshape(F, 1, B)
    dense_t = dense_features.T.reshape(D, 1, B)
    tail_flat = W_cat[:, VA:].reshape(F * VT)
    flat_w, ids_fm, dense_fm = _flatten_all(W_cat, tail_flat, ids_t, dense_t)
    q = jnp.arange(IPW, dtype=jnp.int32)
    oc = (q // EPW) * VP
    ot = TAIL - VA + (q // EPW) * VT
    wd_bc = jnp.broadcast_to(W_dense.reshape(D, 1), (D, LANE))
    bias_bc = jnp.broadcast_to(bias.reshape(1), (LANE,))
    out = _sc_main(ids_fm, dense_fm, flat_w, oc, ot, wd_bc, bias_bc)
    return out.reshape(B, 1)
